# baseline (speedup 1.0000x reference)
import functools

import jax
import jax.numpy as jnp
from jax import lax
from jax.experimental import pallas as pl
from jax.experimental.pallas import tpu as pltpu
from jax.experimental.pallas import tpu_sc as plsc

_D = 32
_LANES = 16
_KB = 512


def kernel(x, table):
    B0, S = x.shape
    V, D = table.shape
    xP = x.T.reshape(S // 8, 8, B0 // 128, 128).transpose(0, 2, 1, 3)
    tT = table.T

    info = plsc.get_sparse_core_info()
    NC, NS = info.num_cores, info.num_subcores
    NW = NC * NS
    CH = B0 // NW
    NA = S // 8
    n_full = V // _KB
    tail = V - n_full * _KB
    assert CH % 128 == 0 and S % 8 == 0 and tail % 16 == 0 and tail > 0

    mesh = plsc.VectorSubcoreMesh(core_axis_name="c", subcore_axis_name="s")

    @functools.partial(
        pl.kernel,
        mesh=mesh,
        out_type=(
            jax.ShapeDtypeStruct((S, D, B0), jnp.float32),
            jax.ShapeDtypeStruct((V, D), jnp.float32),
        ),
        scratch_types=[
            pltpu.VMEM((4, 8, 128), jnp.int32),
            pltpu.VMEM((CH,), jnp.int32),
            pltpu.VMEM((CH,), jnp.int32),
            pltpu.VMEM((CH, _D), jnp.float32),
            pltpu.VMEM((CH, _D), jnp.float32),
            pltpu.VMEM((_D, CH + 1), jnp.float32),
            pltpu.VMEM((_D, CH + 1), jnp.float32),
            pltpu.VMEM((_D, _KB), jnp.float32),
            pltpu.VMEM((_KB, _D + 1), jnp.float32),
            pltpu.SemaphoreType.DMA,
            pltpu.SemaphoreType.DMA,
            pltpu.SemaphoreType.DMA,
            pltpu.SemaphoreType.DMA,
        ],
        compiler_params=pltpu.CompilerParams(
            use_tc_tiling_on_sc=False, needs_layout_passes=False),
    )
    def emb(xp_hbm, tt_hbm, out_hbm, st_hbm, xa_v, idx_v0, idx_v1,
            rows_v0, rows_v1, tr_v0, tr_v1, sl_v, so_v,
            gsem0, gsem1, wsem0, wsem1):
        idxs = (idx_v0, idx_v1)
        rows = (rows_v0, rows_v1)
        trs = (tr_v0, tr_v1)
        gsems = (gsem0, gsem1)
        wsems = (wsem0, wsem1)
        sid = lax.axis_index("s")
        wid = sid * NC + lax.axis_index("c")
        ioff = wid * CH
        b0 = wid * (CH // 128)
        lane = lax.iota(jnp.int32, _LANES)
        hi = lane + _LANES

        def stage_block(r0, nrows):
            pltpu.sync_copy(tt_hbm.at[:, pl.ds(r0, nrows)],
                            sl_v.at[:, pl.ds(0, nrows)])

            @plsc.parallel_loop(0, _D)
            def tpose(c):
                col = jnp.full((_LANES,), c, jnp.int32)
                for k0 in range(0, nrows, _LANES):
                    v = sl_v[c, pl.ds(k0, _LANES)]
                    plsc.store_scatter(so_v, [k0 + lane, col], v)

            pltpu.sync_copy(so_v.at[pl.ds(0, nrows), pl.ds(0, _D)],
                            st_hbm.at[pl.ds(r0, nrows)])

        def stage_iter(i, carry):
            blk = sid + i * NS

            @pl.when(blk < n_full)
            def _():
                stage_block(blk * _KB, _KB)

            return carry

        n_iter = (n_full + NS - 1) // NS
        lax.fori_loop(0, n_iter, stage_iter, 0,
                      unroll=False)

        @pl.when(sid == NS - 1)
        def _():
            stage_block(n_full * _KB, tail)

        plsc.subcore_barrier()

        def load_xa(a):
            pltpu.sync_copy(xp_hbm.at[a, pl.ds(b0, CH // 128)], xa_v)

        def prep(s8, b):
            @plsc.parallel_loop(0, CH // _LANES, unroll=4)
            def one(i):
                k = i // 8
                j = i % 8
                v = xa_v[k, s8, pl.ds(j * _LANES, _LANES)]
                ok = (v >= 0) & (v < V)
                idxs[b][pl.ds(i * _LANES, _LANES)] = jnp.where(ok, v, 0)

        def gather_start(b):
            pltpu.async_copy(st_hbm.at[idxs[b]], rows[b], gsems[b])

        def gather_wait(b):
            pltpu.make_async_copy(st_hbm.at[idxs[b]], rows[b],
                                  gsems[b]).wait()

        def transpose(b):
            @plsc.parallel_loop(0, CH, unroll=8)
            def one(j):
                col = jnp.full((_LANES,), j, jnp.int32)
                va = rows[b][j, pl.ds(0, _LANES)]
                vb = rows[b][j, pl.ds(_LANES, _LANES)]
                plsc.store_scatter(trs[b], [lane, col], va)
                plsc.store_scatter(trs[b], [hi, col], vb)

        def write_start(s, b):
            pltpu.async_copy(trs[b].at[:, pl.ds(0, CH)],
                             out_hbm.at[s, :, pl.ds(ioff, CH)], wsems[b])

        def write_wait(b):
            pltpu.make_async_copy(trs[b].at[:, pl.ds(0, CH)],
                                  out_hbm.at[0, :, pl.ds(ioff, CH)],
                                  wsems[b]).wait()

        def chunk(a, s8, b, w_ok):
            s = a * 8 + s8
            prep(s8, b)
            gather_start(b)
            gather_wait(1 - b)
            if w_ok:
                write_wait(1 - b)
            transpose(1 - b)
            write_start(s - 1, 1 - b)

        load_xa(0)
        prep(0, 0)
        gather_start(0)
        prep(1, 1)
        gather_start(1)
        gather_wait(0)
        transpose(0)
        write_start(0, 0)
        for s8 in range(2, 8):
            chunk(0, s8, s8 % 2, s8 >= 3)

        def ablock(a, carry):
            load_xa(a)
            for s8 in range(8):
                chunk(a, s8, s8 % 2, True)
            return carry

        lax.fori_loop(1, NA, ablock, 0)

        gather_wait(1)
        write_wait(1)
        transpose(1)
        write_start(S - 1, 1)
        write_wait(0)
        write_wait(1)

    out, _ = emb(xP, tT)
    return out.transpose(2, 0, 1)


# speedup vs baseline: 3.1581x; 3.1581x over previous
import functools

import jax
import jax.numpy as jnp
from jax import lax
from jax.experimental import pallas as pl
from jax.experimental.pallas import tpu as pltpu
from jax.experimental.pallas import tpu_sc as plsc

_D = 32
_LANES = 16
_KB = 512


def kernel(x, table):
    B0, S = x.shape
    V, D = table.shape
    xP = x.T.reshape(S // 8, 8, B0 // 128, 128).transpose(0, 2, 1, 3)

    info = plsc.get_sparse_core_info()
    NC, NS = info.num_cores, info.num_subcores
    NW = NC * NS
    CH = B0 // NW
    NA = S // 8
    assert CH % 128 == 0 and S % 8 == 0

    mesh = plsc.VectorSubcoreMesh(core_axis_name="c", subcore_axis_name="s")

    @functools.partial(
        pl.kernel,
        mesh=mesh,
        out_type=jax.ShapeDtypeStruct((S, D, B0), jnp.float32),
        scratch_types=[
            pltpu.VMEM((4, 8, 128), jnp.int32),
            pltpu.VMEM((CH,), jnp.int32),
            pltpu.VMEM((CH,), jnp.int32),
            pltpu.VMEM((CH, _D), jnp.float32),
            pltpu.VMEM((CH, _D), jnp.float32),
            pltpu.VMEM((_D, CH + 1), jnp.float32),
            pltpu.VMEM((_D, CH + 1), jnp.float32),
            pltpu.SemaphoreType.DMA,
            pltpu.SemaphoreType.DMA,
            pltpu.SemaphoreType.DMA,
            pltpu.SemaphoreType.DMA,
        ],
        compiler_params=pltpu.CompilerParams(
            use_tc_tiling_on_sc=False, needs_layout_passes=False),
    )
    def emb(xp_hbm, st_hbm, out_hbm, xa_v, idx_v0, idx_v1,
            rows_v0, rows_v1, tr_v0, tr_v1,
            gsem0, gsem1, wsem0, wsem1):
        idxs = (idx_v0, idx_v1)
        rows = (rows_v0, rows_v1)
        trs = (tr_v0, tr_v1)
        gsems = (gsem0, gsem1)
        wsems = (wsem0, wsem1)
        sid = lax.axis_index("s")
        wid = sid * NC + lax.axis_index("c")
        ioff = wid * CH
        b0 = wid * (CH // 128)
        lane = lax.iota(jnp.int32, _LANES)
        hi = lane + _LANES

        def load_xa(a):
            pltpu.sync_copy(xp_hbm.at[a, pl.ds(b0, CH // 128)], xa_v)

        def prep(s8, b):
            @plsc.parallel_loop(0, CH // _LANES, unroll=4)
            def one(i):
                k = i // 8
                j = i % 8
                v = xa_v[k, s8, pl.ds(j * _LANES, _LANES)]
                ok = (v >= 0) & (v < V)
                idxs[b][pl.ds(i * _LANES, _LANES)] = jnp.where(ok, v, 0)

        def gather_start(b):
            pltpu.async_copy(st_hbm.at[idxs[b]], rows[b], gsems[b])

        def gather_wait(b):
            pltpu.make_async_copy(st_hbm.at[idxs[b]], rows[b],
                                  gsems[b]).wait()

        def transpose(b):
            @plsc.parallel_loop(0, CH, unroll=8)
            def one(j):
                col = jnp.full((_LANES,), j, jnp.int32)
                va = rows[b][j, pl.ds(0, _LANES)]
                vb = rows[b][j, pl.ds(_LANES, _LANES)]
                plsc.store_scatter(trs[b], [lane, col], va)
                plsc.store_scatter(trs[b], [hi, col], vb)

        def write_start(s, b):
            pltpu.async_copy(trs[b].at[:, pl.ds(0, CH)],
                             out_hbm.at[s, :, pl.ds(ioff, CH)], wsems[b])

        def write_wait(b):
            pltpu.make_async_copy(trs[b].at[:, pl.ds(0, CH)],
                                  out_hbm.at[0, :, pl.ds(ioff, CH)],
                                  wsems[b]).wait()

        def chunk(a, s8, b, w_ok):
            s = a * 8 + s8
            prep(s8, b)
            gather_start(b)
            gather_wait(1 - b)
            if w_ok:
                write_wait(1 - b)
            transpose(1 - b)
            write_start(s - 1, 1 - b)

        load_xa(0)
        prep(0, 0)
        gather_start(0)
        prep(1, 1)
        gather_start(1)
        gather_wait(0)
        transpose(0)
        write_start(0, 0)
        for s8 in range(2, 8):
            chunk(0, s8, s8 % 2, s8 >= 3)

        def ablock(a, carry):
            load_xa(a)
            for s8 in range(8):
                chunk(a, s8, s8 % 2, True)
            return carry

        lax.fori_loop(1, NA, ablock, 0)

        gather_wait(1)
        write_wait(1)
        transpose(1)
        write_start(S - 1, 1)
        write_wait(0)
        write_wait(1)

    out = emb(xP, table)
    return out.transpose(2, 0, 1)
